# baseline (speedup 1.0000x reference)
import functools

import jax
import jax.numpy as jnp
from jax import lax
from jax.experimental import pallas as pl
from jax.experimental.pallas import tpu as pltpu
from jax.experimental.pallas import tpu_sc as plsc

_BATCH = 16384
_NF = 26
_D = 64
_NW = 32
_JB = _BATCH // (128 * _NW)
_NCHUNK = _JB * _NF


def _emb_body(idx_hbm, table_hbm, out_hbm, slab_v, idx_all, rows_v, tbuf_v,
              gs0, gs1, os0, os1):
    gsem = (gs0, gs1)
    osem = (os0, os1)
    wid = lax.axis_index("s") * 2 + lax.axis_index("c")
    iota16 = lax.iota(jnp.int32, 16)

    def stage(jj, carry):
        pltpu.sync_copy(idx_hbm.at[pl.ds((wid * _JB + jj) * 128, 128)], slab_v)

        def extract(f, carry2):
            c = jj * _NF + f
            fvec = jnp.full((16,), f, jnp.int32)
            for k in range(8):
                v = plsc.load_gather(slab_v, [iota16 + 16 * k, fvec])
                idx_all[c, pl.ds(16 * k, 16)] = v
            return carry2

        lax.fori_loop(0, _NF, extract, 0)
        return carry

    lax.fori_loop(0, _JB, stage, 0)

    def gather(c, b):
        return pltpu.make_async_copy(
            table_hbm.at[idx_all.at[c]], rows_v.at[b], gsem[b]
        )

    def outwrite(c, b):
        jj = c // _NF
        f = c % _NF
        bb = (wid * _JB + jj) * 128
        return pltpu.make_async_copy(
            tbuf_v.at[b], out_hbm.at[f, :, pl.ds(bb, 128)], osem[b]
        )

    def transpose(b):
        rows_b = rows_v.at[b]

        def trans(d, carry):
            dvec = jnp.full((16,), d, jnp.int32)
            for k in range(8):
                v = plsc.load_gather(rows_b, [iota16 + 16 * k, dvec])
                tbuf_v[b, d, pl.ds(16 * k, 16)] = v
            return carry

        lax.fori_loop(0, _D, trans, 0)

    gather(0, 0).start()

    def body(g, carry):
        for b in range(2):
            c = g * 2 + b
            cn = c + 1

            @pl.when(cn < _NCHUNK)
            def _():
                gather(cn, 1 - b).start()

            gather(c, b).wait()

            @pl.when(c >= 2)
            def _():
                outwrite(c - 2, b).wait()

            transpose(b)
            outwrite(c, b).start()
        return carry

    lax.fori_loop(0, _NCHUNK // 2, body, 0)
    outwrite(_NCHUNK - 2, 0).wait()
    outwrite(_NCHUNK - 1, 1).wait()


@jax.jit
def kernel(indices, table):
    mesh = plsc.VectorSubcoreMesh(core_axis_name="c", subcore_axis_name="s")
    run = functools.partial(
        pl.kernel,
        out_type=jax.ShapeDtypeStruct((_NF, _D, _BATCH), jnp.float32),
        mesh=mesh,
        scratch_types=[
            pltpu.VMEM((128, _NF), jnp.int32),
            pltpu.VMEM((_NCHUNK, 128), jnp.int32),
            pltpu.VMEM((2, 128, _D), jnp.float32),
            pltpu.VMEM((2, _D, 128), jnp.float32),
        ]
        + [pltpu.SemaphoreType.DMA] * 4,
        compiler_params=pltpu.CompilerParams(
            use_tc_tiling_on_sc=False, needs_layout_passes=False
        ),
    )(_emb_body)
    out_t = run(indices, table)
    return jnp.transpose(out_t, (2, 0, 1))


# speedup vs baseline: 1.4302x; 1.4302x over previous
import functools

import jax
import jax.numpy as jnp
from jax import lax
from jax.experimental import pallas as pl
from jax.experimental.pallas import tpu as pltpu
from jax.experimental.pallas import tpu_sc as plsc

_BATCH = 16384
_NF = 26
_D = 64
_NW = 32
_BPW = _BATCH // _NW
_QB = _BPW // 128
_NCHUNK = _QB * _NF
_NBUF = 4


def _emb_body(idxt_hbm, table_hbm, out_hbm, idx_v, rows_v, *sems):
    gsem = sems[:_NBUF]
    osem = sems[_NBUF:]
    wid = lax.axis_index("s") * 2 + lax.axis_index("c")
    b0 = wid * _BPW
    pltpu.sync_copy(idxt_hbm.at[:, pl.ds(b0, _BPW)], idx_v)

    def gather(c, b):
        q = c // _NF
        f = c % _NF
        return pltpu.make_async_copy(
            table_hbm.at[idx_v.at[f, pl.ds(q * 128, 128)]],
            rows_v.at[b],
            gsem[b],
        )

    def outwrite(c, b):
        q = c // _NF
        f = c % _NF
        return pltpu.make_async_copy(
            rows_v.at[b],
            out_hbm.at[pl.ds(b0 + q * 128, 128), f],
            osem[b],
        )

    for b in range(_NBUF):
        gather(b, b).start()

    def body(g, carry):
        g0 = g * _NBUF
        for b in range(_NBUF):
            c = g0 + b
            gather(c, b).wait()
            outwrite(c, b).start()
        for b in range(_NBUF):
            cn = g0 + _NBUF + b

            @pl.when(cn < _NCHUNK)
            def _():
                outwrite(cn - _NBUF, b).wait()
                gather(cn, b).start()
        return carry

    lax.fori_loop(0, _NCHUNK // _NBUF, body, 0)

    for b in range(_NBUF):
        outwrite(_NCHUNK - _NBUF + b, b).wait()


@jax.jit
def kernel(indices, table):
    mesh = plsc.VectorSubcoreMesh(core_axis_name="c", subcore_axis_name="s")
    run = functools.partial(
        pl.kernel,
        out_type=jax.ShapeDtypeStruct((_BATCH, _NF, _D), jnp.float32),
        mesh=mesh,
        scratch_types=[
            pltpu.VMEM((_NF, _BPW), jnp.int32),
            pltpu.VMEM((_NBUF, 128, _D), jnp.float32),
        ]
        + [pltpu.SemaphoreType.DMA] * (2 * _NBUF),
        compiler_params=pltpu.CompilerParams(use_tc_tiling_on_sc=False),
    )(_emb_body)
    return run(indices.T, table)
